# speedup vs baseline: 1.2282x; 1.2282x over previous
import jax
import jax.numpy as jnp
from jax.experimental import pallas as pl
from jax.experimental.pallas import tpu as pltpu

_FEATURE_SIZE = 256
_HORIZON = 100.0
_NUM_SEGS = 16
_BLK = 512


def _body(cu_ref, coeff_ref, tok_ref, out_ref):
    i = pl.program_id(0)
    base = i * _BLK
    rows = jax.lax.broadcasted_iota(jnp.int32, (_BLK, 1), 0) + base
    off = jnp.zeros((_BLK, 1), jnp.int32)
    for j in range(1, _NUM_SEGS):
        b = cu_ref[j]
        off = jnp.maximum(off, jnp.where(rows >= b, b, 0))
    pos = (rows - off).astype(jnp.float32)
    z = pos * coeff_ref[...]
    out_ref[...] = tok_ref[...] + jnp.sin(z)


@jax.jit
def kernel(tokens, cu_seqlens):
    total, size = tokens.shape
    idx = jnp.arange(size, dtype=jnp.float32)
    parity = jnp.mod(idx, 2.0)
    freq = 1.0 / (_HORIZON ** ((idx - parity) / size))
    coeff = (freq + (jnp.pi / 2.0) * parity).reshape(1, size)

    grid = (total // _BLK,)
    return pl.pallas_call(
        _body,
        grid_spec=pltpu.PrefetchScalarGridSpec(
            num_scalar_prefetch=1,
            grid=grid,
            in_specs=[
                pl.BlockSpec((1, size), lambda i, cu: (0, 0)),
                pl.BlockSpec((_BLK, size), lambda i, cu: (i, 0)),
            ],
            out_specs=pl.BlockSpec((_BLK, size), lambda i, cu: (i, 0)),
        ),
        out_shape=jax.ShapeDtypeStruct((total, size), jnp.float32),
        compiler_params=pltpu.CompilerParams(
            dimension_semantics=("arbitrary",),
        ),
    )(cu_seqlens, coeff, tokens)


# speedup vs baseline: 2.2807x; 1.8570x over previous
import jax
import jax.numpy as jnp
from jax.experimental import pallas as pl
from jax.experimental.pallas import tpu as pltpu

_HORIZON = 100.0
_NUM_SEGS = 16
_BLK = 512

_INV_PIO2 = 0.6366197466850281
_MAGIC = 12582912.0
_C1 = 1.5703125
_C2 = 0.000484466552734375
_C3 = -6.407499313354492e-07
_C4 = 9.92093629470503e-10
_S1, _S2, _S3 = -1.6666654611e-1, 8.3321608736e-3, -1.9515295891e-4
_K1, _K2, _K3 = -0.5, 4.166664568298827e-2, -1.388731625493765e-3


def _fast_sin(z):
    u = z * _INV_PIO2 + _MAGIC
    nf = u - _MAGIC
    ub = jax.lax.bitcast_convert_type(u, jnp.int32)
    r = z - nf * _C1
    r = r - nf * _C2
    r = r - nf * _C3
    r = r - nf * _C4
    y = r * r
    swap = (ub & 1) == 1
    a3 = jnp.where(swap, _K3, _S3)
    a2 = jnp.where(swap, _K2, _S2)
    a1 = jnp.where(swap, _K1, _S1)
    p = (a3 * y + a2) * y + a1
    p = p * y + 1.0
    h = jnp.where(swap, 1.0, r)
    sgn = 1.0 - (ub & 2).astype(jnp.float32)
    return h * p * sgn


def _body(cu_ref, coeff_ref, tok_ref, out_ref):
    i = pl.program_id(0)
    base = i * _BLK
    rows = jax.lax.broadcasted_iota(jnp.int32, (_BLK, 1), 0) + base
    off = jnp.zeros((_BLK, 1), jnp.int32)
    for j in range(1, _NUM_SEGS):
        b = cu_ref[j]
        off = jnp.maximum(off, jnp.where(rows >= b, b, 0))
    pos = (rows - off).astype(jnp.float32)
    z = pos * coeff_ref[...]
    out_ref[...] = tok_ref[...] + _fast_sin(z)


@jax.jit
def kernel(tokens, cu_seqlens):
    total, size = tokens.shape
    idx = jnp.arange(size, dtype=jnp.float32)
    parity = jnp.mod(idx, 2.0)
    freq = 1.0 / (_HORIZON ** ((idx - parity) / size))
    coeff = (freq + (jnp.pi / 2.0) * parity).reshape(1, size)

    grid = (total // _BLK,)
    return pl.pallas_call(
        _body,
        grid_spec=pltpu.PrefetchScalarGridSpec(
            num_scalar_prefetch=1,
            grid=grid,
            in_specs=[
                pl.BlockSpec((1, size), lambda i, cu: (0, 0)),
                pl.BlockSpec((_BLK, size), lambda i, cu: (i, 0)),
            ],
            out_specs=pl.BlockSpec((_BLK, size), lambda i, cu: (i, 0)),
        ),
        out_shape=jax.ShapeDtypeStruct((total, size), jnp.float32),
        compiler_params=pltpu.CompilerParams(
            dimension_semantics=("arbitrary",),
        ),
    )(cu_seqlens, coeff, tokens)


# speedup vs baseline: 2.6220x; 1.1496x over previous
import jax
import jax.numpy as jnp
from jax.experimental import pallas as pl
from jax.experimental.pallas import tpu as pltpu

_HORIZON = 100.0
_NUM_SEGS = 16
_BLK = 512

_INV_PIO2 = 0.6366197466850281
_MAGIC = 12582912.0
_C1 = 1.5703125
_C2 = 0.000484466552734375
_C3 = -6.407499313354492e-07
_S1, _S2, _S3 = -1.6666654611e-1, 8.3321608736e-3, -1.9515295891e-4
_K1, _K2, _K3 = -0.5, 4.166664568298827e-2, -1.388731625493765e-3


def _fast_sin(z):
    u = z * _INV_PIO2 + _MAGIC
    nf = u - _MAGIC
    ub = jax.lax.bitcast_convert_type(u, jnp.int32)
    r = z - nf * _C1
    r = r - nf * _C2
    r = r - nf * _C3
    y = r * r
    swap = (ub & 1) == 1
    a3 = jnp.where(swap, _K3, _S3)
    a2 = jnp.where(swap, _K2, _S2)
    a1 = jnp.where(swap, _K1, _S1)
    p = (a3 * y + a2) * y + a1
    p = p * y + 1.0
    h = jnp.where(swap, 1.0, r)
    hp = jax.lax.bitcast_convert_type(h * p, jnp.int32)
    return jax.lax.bitcast_convert_type(hp ^ ((ub & 2) << 30), jnp.float32)


def _body(cu_ref, coeff_ref, tok_ref, out_ref, pos_ref):
    i = pl.program_id(0)
    base = i * _BLK
    rows = jax.lax.broadcasted_iota(jnp.int32, (_BLK, 1), 0) + base
    base_off = jnp.int32(0)
    anyin = jnp.bool_(False)
    for j in range(1, _NUM_SEGS):
        b = cu_ref[j]
        base_off = jnp.maximum(base_off, jnp.where(b <= base, b, 0))
        anyin = anyin | ((b > base) & (b < base + _BLK))

    @pl.when(jnp.logical_not(anyin))
    def _():
        pos_ref[...] = (rows - base_off).astype(jnp.float32)

    @pl.when(anyin)
    def _():
        off = jnp.full((_BLK, 1), base_off, jnp.int32)
        for j in range(1, _NUM_SEGS):
            b = cu_ref[j]
            off = jnp.maximum(off, jnp.where(rows >= b, b, 0))
        pos_ref[...] = (rows - off).astype(jnp.float32)

    z = pos_ref[...] * coeff_ref[...]
    out_ref[...] = tok_ref[...] + _fast_sin(z)


@jax.jit
def kernel(tokens, cu_seqlens):
    total, size = tokens.shape
    idx = jnp.arange(size, dtype=jnp.float32)
    parity = jnp.mod(idx, 2.0)
    freq = 1.0 / (_HORIZON ** ((idx - parity) / size))
    coeff = (freq + (jnp.pi / 2.0) * parity).reshape(1, size)

    grid = (total // _BLK,)
    return pl.pallas_call(
        _body,
        grid_spec=pltpu.PrefetchScalarGridSpec(
            num_scalar_prefetch=1,
            grid=grid,
            in_specs=[
                pl.BlockSpec((1, size), lambda i, cu: (0, 0)),
                pl.BlockSpec((_BLK, size), lambda i, cu: (i, 0)),
            ],
            out_specs=pl.BlockSpec((_BLK, size), lambda i, cu: (i, 0)),
            scratch_shapes=[pltpu.VMEM((_BLK, 1), jnp.float32)],
        ),
        out_shape=jax.ShapeDtypeStruct((total, size), jnp.float32),
        compiler_params=pltpu.CompilerParams(
            dimension_semantics=("arbitrary",),
        ),
    )(cu_seqlens, coeff, tokens)


# speedup vs baseline: 4.3240x; 1.6491x over previous
import jax
import jax.numpy as jnp
from jax.experimental import pallas as pl
from jax.experimental.pallas import tpu as pltpu

_HORIZON = 100.0
_NUM_SEGS = 16
_BLK = 2048

_INV_PIO2 = 0.6366197466850281
_MAGIC = 12582912.0
_C1 = 1.5703125
_C2 = 0.000484466552734375
_C3 = -6.407499313354492e-07
_S1, _S2, _S3 = -1.6666654611e-1, 8.3321608736e-3, -1.9515295891e-4
_K1, _K2, _K3 = -0.5, 4.166664568298827e-2, -1.388731625493765e-3


def _fast_sin(z):
    u = z * _INV_PIO2 + _MAGIC
    nf = u - _MAGIC
    ub = jax.lax.bitcast_convert_type(u, jnp.int32)
    r = z - nf * _C1
    r = r - nf * _C2
    r = r - nf * _C3
    y = r * r
    swap = (ub & 1) == 1
    a3 = jnp.where(swap, _K3, _S3)
    a2 = jnp.where(swap, _K2, _S2)
    a1 = jnp.where(swap, _K1, _S1)
    p = (a3 * y + a2) * y + a1
    p = p * y + 1.0
    h = jnp.where(swap, 1.0, r)
    hp = jax.lax.bitcast_convert_type(h * p, jnp.int32)
    return jax.lax.bitcast_convert_type(hp ^ ((ub & 2) << 30), jnp.float32)


def _body(cu_ref, coeff_ref, tok_ref, out_ref, pos_ref, bl_ref):
    i = pl.program_id(0)
    base = i * _BLK
    rows = jax.lax.broadcasted_iota(jnp.int32, (_BLK, 1), 0) + base
    base_off = jnp.int32(0)
    cnt = jnp.int32(0)
    for j in range(1, _NUM_SEGS):
        b = cu_ref[j]
        base_off = jnp.maximum(base_off, jnp.where(b <= base, b, 0))
        inb = (b > base) & (b < base + _BLK)

        @pl.when(inb)
        def _(b=b, cnt=cnt):
            bl_ref[cnt] = b

        cnt = cnt + inb.astype(jnp.int32)

    pos_ref[...] = (rows - base_off).astype(jnp.float32)

    def _fixup(k, carry):
        b = bl_ref[k]
        pos_ref[...] = jnp.where(
            rows >= b, (rows - b).astype(jnp.float32), pos_ref[...])
        return carry

    jax.lax.fori_loop(0, cnt, _fixup, 0)

    z = pos_ref[...] * coeff_ref[...]
    out_ref[...] = tok_ref[...] + _fast_sin(z)


@jax.jit
def kernel(tokens, cu_seqlens):
    total, size = tokens.shape
    idx = jnp.arange(size, dtype=jnp.float32)
    parity = jnp.mod(idx, 2.0)
    freq = 1.0 / (_HORIZON ** ((idx - parity) / size))
    coeff = (freq + (jnp.pi / 2.0) * parity).reshape(1, size)

    grid = (total // _BLK,)
    return pl.pallas_call(
        _body,
        grid_spec=pltpu.PrefetchScalarGridSpec(
            num_scalar_prefetch=1,
            grid=grid,
            in_specs=[
                pl.BlockSpec((1, size), lambda i, cu: (0, 0)),
                pl.BlockSpec((_BLK, size), lambda i, cu: (i, 0)),
            ],
            out_specs=pl.BlockSpec((_BLK, size), lambda i, cu: (i, 0)),
            scratch_shapes=[
                pltpu.VMEM((_BLK, 1), jnp.float32),
                pltpu.SMEM((_NUM_SEGS,), jnp.int32),
            ],
        ),
        out_shape=jax.ShapeDtypeStruct((total, size), jnp.float32),
        compiler_params=pltpu.CompilerParams(
            dimension_semantics=("arbitrary",),
        ),
    )(cu_seqlens, coeff, tokens)
